# Initial kernel scaffold; baseline (speedup 1.0000x reference)
import jax
import jax.numpy as jnp
from jax.experimental import pallas as pl


def kernel(input, table):
    raise NotImplementedError("write your pallas kernel here")


# speedup vs baseline: 2.4190x; 2.4190x over previous
import functools
import math

import jax
import jax.numpy as jnp
from jax import lax
from jax.experimental import pallas as pl
from jax.experimental.pallas import tpu as pltpu
from jax.experimental.pallas import tpu_sc as plsc

_NW = 32
_CHUNK = 128
_LANES = 16


def _emb_body(idx_hbm, table_hbm, out_hbm, idx_v, rows_v, sem, *, n_chunks, d):
    wid = lax.axis_index("s") * 2 + lax.axis_index("c")
    pltpu.sync_copy(idx_hbm.at[wid], idx_v)
    scale = jnp.full((_LANES,), math.sqrt(d), dtype=jnp.float32)
    vecs_per_row = d // _LANES

    def chunk_body(j, _):
        pltpu.async_copy(table_hbm.at[idx_v.at[j]], rows_v, sem).wait()

        def row_body(i, _):
            for v in range(vecs_per_row):
                sl = pl.ds(v * _LANES, _LANES)
                rows_v[i, sl] = rows_v[i, sl] * scale
            return 0

        lax.fori_loop(0, _CHUNK, row_body, 0)
        pltpu.sync_copy(rows_v, out_hbm.at[wid, j])
        return 0

    lax.fori_loop(0, n_chunks, chunk_body, 0)


def kernel(input, table):
    b, l = input.shape
    v, d = table.shape
    n = b * l
    assert n % (_NW * _CHUNK) == 0
    n_chunks = n // (_NW * _CHUNK)

    idx = input.reshape(_NW, n_chunks, _CHUNK).astype(jnp.int32)
    mesh = plsc.VectorSubcoreMesh(core_axis_name="c", subcore_axis_name="s")

    emb = pl.kernel(
        functools.partial(_emb_body, n_chunks=n_chunks, d=d),
        mesh=mesh,
        out_type=jax.ShapeDtypeStruct((_NW, n_chunks, _CHUNK, d), jnp.float32),
        scratch_types=[
            pltpu.VMEM((n_chunks, _CHUNK), jnp.int32),
            pltpu.VMEM((_CHUNK, d), jnp.float32),
            pltpu.SemaphoreType.DMA,
        ],
    )(idx, table)
    return emb.reshape(b, l, d)


# speedup vs baseline: 2.8902x; 1.1948x over previous
import functools
import math

import jax
import jax.numpy as jnp
from jax import lax
from jax.experimental import pallas as pl
from jax.experimental.pallas import tpu as pltpu
from jax.experimental.pallas import tpu_sc as plsc

_NW = 32
_CHUNK = 128
_LANES = 16


def _emb_body(idx_hbm, table_hbm, out_hbm, idx_v, buf0, buf1, g0, g1, w0, w1,
              *, n_chunks, d):
    wid = lax.axis_index("s") * 2 + lax.axis_index("c")
    pltpu.sync_copy(idx_hbm.at[wid], idx_v)
    scale = jnp.full((_LANES,), math.sqrt(d), dtype=jnp.float32)
    vecs_per_row = d // _LANES
    bufs = (buf0, buf1)
    gsem = (g0, g1)
    wsem = (w0, w1)

    def start_gather(j, b):
        pltpu.async_copy(table_hbm.at[idx_v.at[j]], bufs[b], gsem[b])

    def wait_gather(b):
        pltpu.make_async_copy(table_hbm.at[idx_v.at[0]], bufs[b], gsem[b]).wait()

    def start_write(j, b):
        pltpu.async_copy(bufs[b], out_hbm.at[wid, j], wsem[b])

    def wait_write(b):
        pltpu.make_async_copy(bufs[b], out_hbm.at[wid, 0], wsem[b]).wait()

    def scale_buf(b):
        buf = bufs[b]

        def row_body(i, _):
            r = i * 2
            for rr in (r, r + 1):
                for v in range(vecs_per_row):
                    sl = pl.ds(v * _LANES, _LANES)
                    buf[rr, sl] = buf[rr, sl] * scale
            return 0

        lax.fori_loop(0, _CHUNK // 2, row_body, 0, unroll=2)

    def steady(j, b):
        wait_write(1 - b)
        start_gather(j + 1, 1 - b)
        wait_gather(b)
        scale_buf(b)
        start_write(j, b)

    start_gather(0, 0)
    start_gather(1, 1)
    wait_gather(0)
    scale_buf(0)
    start_write(0, 0)

    def outer(k, _):
        j0 = 1 + 2 * k
        steady(j0, 1)
        steady(j0 + 1, 0)
        return 0

    lax.fori_loop(0, (n_chunks - 2) // 2, outer, 0)

    wait_gather(1)
    scale_buf(1)
    start_write(n_chunks - 1, 1)
    wait_write(0)
    wait_write(1)


def kernel(input, table):
    b, l = input.shape
    v, d = table.shape
    n = b * l
    assert n % (_NW * _CHUNK) == 0
    n_chunks = n // (_NW * _CHUNK)
    assert n_chunks % 2 == 0

    idx = input.reshape(_NW, n_chunks, _CHUNK).astype(jnp.int32)
    mesh = plsc.VectorSubcoreMesh(core_axis_name="c", subcore_axis_name="s")

    emb = pl.kernel(
        functools.partial(_emb_body, n_chunks=n_chunks, d=d),
        mesh=mesh,
        out_type=jax.ShapeDtypeStruct((_NW, n_chunks, _CHUNK, d), jnp.float32),
        scratch_types=[
            pltpu.VMEM((n_chunks, _CHUNK), jnp.int32),
            pltpu.VMEM((_CHUNK, d), jnp.float32),
            pltpu.VMEM((_CHUNK, d), jnp.float32),
            pltpu.SemaphoreType.DMA,
            pltpu.SemaphoreType.DMA,
            pltpu.SemaphoreType.DMA,
            pltpu.SemaphoreType.DMA,
        ],
    )(idx, table)
    return emb.reshape(b, l, d)


# speedup vs baseline: 2.9459x; 1.0193x over previous
import functools
import math

import jax
import jax.numpy as jnp
from jax import lax
from jax.experimental import pallas as pl
from jax.experimental.pallas import tpu as pltpu
from jax.experimental.pallas import tpu_sc as plsc

_NW = 32
_CHUNK = 128
_LANES = 16
_NBUF = 4


def _emb_body(idx_hbm, table_hbm, out_hbm, idx_v, b0, b1, b2, b3,
              g0, g1, g2, g3, w0, w1, w2, w3, *, n_chunks, d):
    wid = lax.axis_index("s") * 2 + lax.axis_index("c")
    pltpu.sync_copy(idx_hbm.at[wid], idx_v)
    scale = jnp.full((_LANES,), math.sqrt(d), dtype=jnp.float32)
    vecs_per_row = d // _LANES
    bufs = (b0, b1, b2, b3)
    gsem = (g0, g1, g2, g3)
    wsem = (w0, w1, w2, w3)

    def start_gather(j, b):
        pltpu.async_copy(table_hbm.at[idx_v.at[j]], bufs[b], gsem[b])

    def wait_gather(b):
        pltpu.make_async_copy(table_hbm.at[idx_v.at[0]], bufs[b], gsem[b]).wait()

    def start_write(j, b):
        pltpu.async_copy(bufs[b], out_hbm.at[wid, j], wsem[b])

    def wait_write(b):
        pltpu.make_async_copy(bufs[b], out_hbm.at[wid, 0], wsem[b]).wait()

    def scale_buf(b):
        buf = bufs[b]

        def row_body(i, _):
            r = i * 2
            for rr in (r, r + 1):
                for v in range(vecs_per_row):
                    sl = pl.ds(v * _LANES, _LANES)
                    buf[rr, sl] = buf[rr, sl] * scale
            return 0

        lax.fori_loop(0, _CHUNK // 2, row_body, 0, unroll=2)

    def process(j, b, issue_wait, issue_gather):
        wait_gather(b)
        scale_buf(b)
        start_write(j, b)
        if issue_gather:
            g = j + 2
            bg = (b + 2) % _NBUF
            if issue_wait:
                wait_write(bg)
            start_gather(g, bg)

    start_gather(0, 0)
    start_gather(1, 1)
    process(0, 0, issue_wait=False, issue_gather=True)
    process(1, 1, issue_wait=False, issue_gather=True)

    def outer(k, _):
        j0 = 2 + 4 * k
        process(j0, 2, True, True)
        process(j0 + 1, 3, True, True)
        process(j0 + 2, 0, True, True)
        process(j0 + 3, 1, True, True)
        return 0

    n_steady = n_chunks - 4
    lax.fori_loop(0, n_steady // 4, outer, 0)
    j_tail = 2 + (n_steady // 4) * 4
    for t in range(n_steady % 4):
        process(j_tail + t, (j_tail + t) % _NBUF, True, True)

    process(n_chunks - 2, (n_chunks - 2) % _NBUF, False, False)
    process(n_chunks - 1, (n_chunks - 1) % _NBUF, False, False)
    for b in range(_NBUF):
        wait_write(b)


def kernel(input, table):
    b, l = input.shape
    v, d = table.shape
    n = b * l
    assert n % (_NW * _CHUNK) == 0
    n_chunks = n // (_NW * _CHUNK)
    assert n_chunks >= 8

    idx = input.reshape(_NW, n_chunks, _CHUNK).astype(jnp.int32)
    mesh = plsc.VectorSubcoreMesh(core_axis_name="c", subcore_axis_name="s")

    emb = pl.kernel(
        functools.partial(_emb_body, n_chunks=n_chunks, d=d),
        mesh=mesh,
        out_type=jax.ShapeDtypeStruct((_NW, n_chunks, _CHUNK, d), jnp.float32),
        scratch_types=(
            [pltpu.VMEM((n_chunks, _CHUNK), jnp.int32)]
            + [pltpu.VMEM((_CHUNK, d), jnp.float32)] * _NBUF
            + [pltpu.SemaphoreType.DMA] * (2 * _NBUF)
        ),
    )(idx, table)
    return emb.reshape(b, l, d)
